# baseline (speedup 1.0000x reference)
import functools

import jax
import jax.numpy as jnp
from jax import lax
from jax.experimental import pallas as pl
from jax.experimental.pallas import tpu as pltpu
from jax.experimental.pallas import tpu_sc as plsc

N = 100000
E = 1600000
NODE_FIELDS = 8
EDGE_FIELDS = 4
D = 32
H = 64

NC = 2
NS = 16
SUB = 128

HALF = N // NC
TRASH = HALF
ACC_ROWS = 50176
NODE_FULL = HALF // SUB
NODE_TAIL = HALF - NODE_FULL * SUB
NODE_TAIL_BASE = NODE_FULL * SUB
ESUB = E // SUB
ZROWS = 16
ZSUB = ACC_ROWS // ZROWS
EW = EDGE_FIELDS * SUB
NW = NODE_FIELDS * SUB


def _sc_embed_aggregate(node_tables, edge_tables, x_flat, attr_flat, ei_flat):
  mesh = plsc.VectorSubcoreMesh(
      core_axis_name="c", subcore_axis_name="s", num_cores=NC, num_subcores=NS)

  @functools.partial(
      pl.kernel,
      out_type=jax.ShapeDtypeStruct((N, D), jnp.float32),
      mesh=mesh,
      compiler_params=pltpu.CompilerParams(use_tc_tiling_on_sc=False,
                                           needs_layout_passes=False),
      scratch_types=[
          pltpu.VMEM_SHARED((ACC_ROWS, D), jnp.float32),
          pltpu.VMEM((ZROWS, D), jnp.float32),
          pltpu.VMEM((3 * EW, ), jnp.int32),
          pltpu.VMEM((3 * EW, ), jnp.int32),
          pltpu.VMEM((3, SUB), jnp.int32),
          pltpu.VMEM((3, SUB), jnp.int32),
          pltpu.VMEM((SUB,), jnp.int32),
          pltpu.VMEM((NODE_TAIL,), jnp.int32),
          pltpu.VMEM((6, SUB, D), jnp.float32),
          pltpu.SemaphoreType.DMA,
          pltpu.SemaphoreType.DMA,
          pltpu.SemaphoreType.DMA,
          pltpu.SemaphoreType.DMA,
      ],
  )
  def k(node_tbl, edge_tbl, x_h, attr_h, ei_h, out, acc, zbuf, ibuf, qbuf,
        dbuf, dloc, lin, lin_tail, rows, sem_in, sem_g0, sem_g1, sem_s):
    c = lax.axis_index("c")
    s = lax.axis_index("s")
    nbase = c * HALF
    iota = lax.iota(jnp.int32, 16)
    iota4 = iota * EDGE_FIELDS
    iota8 = iota * NODE_FIELDS
    zero16 = jnp.zeros((16,), jnp.float32)

    def zrow(i, _):
      zbuf[i, pl.ds(0, 16)] = zero16
      zbuf[i, pl.ds(16, 16)] = zero16
      return 0
    lax.fori_loop(0, ZROWS, zrow, 0)

    nz = (ZSUB - s + NS - 1) // NS
    def zbody(i, _):
      j = s + i * NS
      pltpu.async_copy(zbuf, acc.at[pl.ds(j * ZROWS, ZROWS)], sem_s)
      return 0
    lax.fori_loop(0, nz, zbody, 0)
    def zdrain(i, _):
      pltpu.make_async_copy(zbuf, acc.at[pl.ds(s * ZROWS, ZROWS)], sem_s).wait()
      return 0
    lax.fori_loop(0, nz, zdrain, 0)

    for kk in range(SUB // 16):
      lin[pl.ds(kk * 16, 16)] = kk * 16 + iota
    for kk in range(NODE_TAIL // 16):
      lin_tail[pl.ds(kk * 16, 16)] = kk * 16 + iota

    plsc.subcore_barrier()

    nn = (NODE_FULL - s + NS - 1) // NS

    def nbody(i, _):
      j = s + i * NS
      b = i & 1
      p0, p1 = 2 * b, 2 * b + 1
      pltpu.async_copy(x_h.at[pl.ds((nbase + j * SUB) * NODE_FIELDS, NW)],
                       ibuf.at[pl.ds(0, NW)], sem_in).wait()
      for f in range(NODE_FIELDS):
        for kk in range(SUB // 16):
          v = plsc.load_gather(ibuf.at[pl.ds(0, NW)],
                               [iota8 + (kk * NODE_FIELDS * 16 + f)])
          qbuf[pl.ds(f * SUB + kk * 16, 16)] = v
      for w in range(NODE_FIELDS // 2):
        add = w > 0
        gcps = [
            pltpu.async_copy(
                node_tbl.at[2 * w].at[qbuf.at[pl.ds(2 * w * SUB, SUB)]],
                rows.at[p0], sem_g0, add=add),
            pltpu.async_copy(
                node_tbl.at[2 * w + 1].at[qbuf.at[pl.ds((2 * w + 1) * SUB, SUB)]],
                rows.at[p1], sem_g0, add=add)]
        for cp in gcps:
          cp.wait()
      @pl.when(i >= 1)
      def _drain_prev():
        for p in (2 * (1 - b), 2 * (1 - b) + 1):
          pltpu.make_async_copy(rows.at[p],
                                acc.at[pl.ds(j * SUB, SUB)].at[lin],
                                sem_s).wait()
      for p in (p0, p1):
        pltpu.async_copy(rows.at[p], acc.at[pl.ds(j * SUB, SUB)].at[lin],
                         sem_s, add=True)
      return 0
    lax.fori_loop(0, nn, nbody, 0)

    blast_n = (nn - 1) & 1
    for p in (2 * blast_n, 2 * blast_n + 1):
      pltpu.make_async_copy(rows.at[p], acc.at[pl.ds(0, SUB)].at[lin],
                            sem_s).wait()

    @pl.when(s == 0)
    def _node_tail():
      pltpu.async_copy(
          x_h.at[pl.ds((nbase + NODE_TAIL_BASE) * NODE_FIELDS,
                       NODE_TAIL * NODE_FIELDS)],
          ibuf.at[pl.ds(0, NODE_TAIL * NODE_FIELDS)], sem_in).wait()
      for f in range(NODE_FIELDS):
        for kk in range(NODE_TAIL // 16):
          v = plsc.load_gather(ibuf.at[pl.ds(0, NODE_TAIL * NODE_FIELDS)],
                               [iota8 + (kk * NODE_FIELDS * 16 + f)])
          qbuf[pl.ds(f * NODE_TAIL + kk * 16, 16)] = v
      for w in range(NODE_FIELDS // 2):
        gcps = [pltpu.async_copy(
            node_tbl.at[2 * w + q].at[qbuf.at[pl.ds((2 * w + q) * NODE_TAIL,
                                                    NODE_TAIL)]],
            rows.at[q, pl.ds(0, NODE_TAIL)], sem_g0, add=w > 0)
            for q in (0, 1)]
        for cp in gcps:
          cp.wait()
      for q in (0, 1):
        pltpu.sync_copy(rows.at[q, pl.ds(0, NODE_TAIL)],
                        acc.at[pl.ds(NODE_TAIL_BASE, NODE_TAIL)].at[lin_tail],
                        add=True)

    ne = (ESUB - s + NS - 1) // NS

    def fire_inputs_e(j, t):
      pltpu.async_copy(attr_h.at[pl.ds(j * EW, EW)],
                       ibuf.at[pl.ds(t * EW, EW)], sem_in)
      pltpu.async_copy(ei_h.at[pl.ds(E + j * SUB, SUB)], dbuf.at[t], sem_in)

    def wait_inputs_e(j, t):
      pltpu.make_async_copy(attr_h.at[pl.ds(j * EW, EW)],
                            ibuf.at[pl.ds(t * EW, EW)], sem_in).wait()
      pltpu.make_async_copy(ei_h.at[pl.ds(E + j * SUB, SUB)], dbuf.at[t],
                            sem_in).wait()

    def prep_e(t):
      for f in range(EDGE_FIELDS):
        for kk in range(SUB // 16):
          v = plsc.load_gather(ibuf.at[pl.ds(t * EW, EW)],
                               [iota4 + (kk * EDGE_FIELDS * 16 + f)])
          qbuf[pl.ds(t * EW + f * SUB + kk * 16, 16)] = v
      for kk in range(SUB // 16):
        v = dbuf[t, pl.ds(kk * 16, 16)]
        loc = v - nbase
        ok = (v >= nbase) & (loc < HALF)
        dloc[t, pl.ds(kk * 16, 16)] = jnp.where(ok, loc, TRASH + kk * 16 + iota)

    def fire_wave0_e(t):
      for q in (0, 1):
        pltpu.async_copy(
            edge_tbl.at[q].at[qbuf.at[pl.ds(t * EW + q * SUB, SUB)]],
            rows.at[2 * t + q], sem_g0)

    def wait_wave0_e(t):
      for q in (0, 1):
        pltpu.make_async_copy(
            edge_tbl.at[q].at[qbuf.at[pl.ds(t * EW + q * SUB, SUB)]],
            rows.at[2 * t + q], sem_g0).wait()

    def fire_wave1_e(t):
      for q in (0, 1):
        pltpu.async_copy(
            edge_tbl.at[2 + q].at[qbuf.at[pl.ds(t * EW + (2 + q) * SUB, SUB)]],
            rows.at[2 * t + q], sem_g1, add=True)

    def wait_wave1_e(t):
      for q in (0, 1):
        pltpu.make_async_copy(
            edge_tbl.at[2 + q].at[qbuf.at[pl.ds(t * EW + (2 + q) * SUB, SUB)]],
            rows.at[2 * t + q], sem_g1).wait()

    def fire_scatter_e(t):
      for q in (0, 1):
        pltpu.async_copy(rows.at[2 * t + q], acc.at[dloc.at[t]], sem_s,
                         add=True)

    def drain_scatter_e(t):
      for q in (0, 1):
        pltpu.make_async_copy(rows.at[2 * t + q], acc.at[dloc.at[t]],
                              sem_s).wait()

    fire_inputs_e(s, 0)
    wait_inputs_e(s, 0)
    prep_e(0)
    fire_wave0_e(0)
    fire_inputs_e(s + NS, 1)

    def ebody(i, _):
      j = s + i * NS
      r = lax.rem(i, 3)
      r_prev = lax.rem(i + 2, 3)
      r_next = lax.rem(i + 1, 3)

      @pl.when(i >= 2)
      def _drain():
        drain_scatter_e(r_next)

      @pl.when(i + 1 < ne)
      def _prep_next():
        wait_inputs_e(j + NS, r_next)
        prep_e(r_next)

      @pl.when(i >= 1)
      def _scatter_prev():
        wait_wave1_e(r_prev)
        fire_scatter_e(r_prev)

      @pl.when(i + 2 < ne)
      def _prefetch():
        fire_inputs_e(j + 2 * NS, r_prev)

      wait_wave0_e(r)
      fire_wave1_e(r)

      @pl.when(i + 1 < ne)
      def _next_wave0():
        fire_wave0_e(r_next)
      return 0
    lax.fori_loop(0, ne, ebody, 0)

    blast = ne - 1
    rl = lax.rem(blast, 3)
    drain_scatter_e(lax.rem(blast + 2, 3))
    wait_wave1_e(rl)
    fire_scatter_e(rl)
    drain_scatter_e(rl)

    plsc.subcore_barrier()

    nx = (NODE_FULL - s + NS - 1) // NS
    def xbody(i, _):
      j = s + i * NS
      pltpu.async_copy(acc.at[pl.ds(j * SUB, SUB)],
                       out.at[pl.ds(nbase + j * SUB, SUB)], sem_s)
      return 0
    lax.fori_loop(0, nx, xbody, 0)
    def xdrain(i, _):
      pltpu.make_async_copy(acc.at[pl.ds(s * SUB, SUB)],
                            out.at[pl.ds(nbase + s * SUB, SUB)], sem_s).wait()
      return 0
    lax.fori_loop(0, nx, xdrain, 0)

    @pl.when(s == 0)
    def _exp_tail():
      pltpu.sync_copy(acc.at[pl.ds(NODE_TAIL_BASE, NODE_TAIL)],
                      out.at[pl.ds(nbase + NODE_TAIL_BASE, NODE_TAIL)])

  return k(node_tables, edge_tables, x_flat, attr_flat, ei_flat)


MLP_BLK = 2000


def _mlp(agg, W_enc, W_dec):
  def body(a_ref, we_ref, wd_ref, o_ref):
    h = jnp.maximum(
        jnp.dot(a_ref[...], we_ref[...], preferred_element_type=jnp.float32),
        0.0)
    o_ref[...] = jnp.dot(h, wd_ref[...], preferred_element_type=jnp.float32)

  return pl.pallas_call(
      body,
      grid=(N // MLP_BLK,),
      in_specs=[
          pl.BlockSpec((MLP_BLK, D), lambda i: (i, 0)),
          pl.BlockSpec((D, H), lambda i: (0, 0)),
          pl.BlockSpec((H, D), lambda i: (0, 0)),
      ],
      out_specs=pl.BlockSpec((MLP_BLK, D), lambda i: (i, 0)),
      out_shape=jax.ShapeDtypeStruct((N, D), jnp.float32),
  )(agg, W_enc, W_dec)


def kernel(x, edge_attr, edge_index, node_tables, edge_tables, W_enc, W_dec):
  agg = _sc_embed_aggregate(node_tables, edge_tables, x.reshape(-1),
                            edge_attr.reshape(-1), edge_index.reshape(-1))
  return _mlp(agg, W_enc, W_dec)


# speedup vs baseline: 2.1889x; 2.1889x over previous
import functools

import jax
import jax.numpy as jnp
from jax import lax
from jax.experimental import pallas as pl
from jax.experimental.pallas import tpu as pltpu
from jax.experimental.pallas import tpu_sc as plsc

N = 100000
E = 1600000
NODE_FIELDS = 8
EDGE_FIELDS = 4
D = 32
H = 64

NC = 2
NS = 16
SUB = 128

HALF = N // NC
TRASH = HALF
ACC_ROWS = 50176
NODE_FULL = HALF // SUB
NODE_TAIL = HALF - NODE_FULL * SUB
NODE_TAIL_BASE = NODE_FULL * SUB
ESUB = E // SUB
ZROWS = 16
ZSUB = ACC_ROWS // ZROWS


def _sc_embed_aggregate(node_tables, edge_tables, xT, attr_i, ei_flat):
  mesh = plsc.VectorSubcoreMesh(
      core_axis_name="c", subcore_axis_name="s", num_cores=NC, num_subcores=NS)

  @functools.partial(
      pl.kernel,
      out_type=jax.ShapeDtypeStruct((N, D), jnp.float32),
      mesh=mesh,
      compiler_params=pltpu.CompilerParams(use_tc_tiling_on_sc=False),
      scratch_types=[
          pltpu.VMEM_SHARED((ACC_ROWS, D), jnp.float32),
          pltpu.VMEM((ZROWS, D), jnp.float32),
          pltpu.VMEM((2, NODE_FIELDS, SUB), jnp.int32),
          pltpu.VMEM((3, EDGE_FIELDS, SUB), jnp.int32),
          pltpu.VMEM((3, SUB), jnp.int32),
          pltpu.VMEM((3, SUB), jnp.int32),
          pltpu.VMEM((2, SUB), jnp.int32),
          pltpu.VMEM((NODE_TAIL,), jnp.int32),
          pltpu.VMEM((6, SUB, D), jnp.float32),
          pltpu.SemaphoreType.DMA,
          pltpu.SemaphoreType.DMA,
          pltpu.SemaphoreType.DMA,
          pltpu.SemaphoreType.DMA,
      ],
  )
  def k(node_tbl, edge_tbl, xT_h, attr_h, ei_h, out, acc, zbuf, xbuf, abuf,
        dbuf, dloc, lin, lin_tail, rows, sem_in, sem_g0, sem_g1, sem_s):
    c = lax.axis_index("c")
    s = lax.axis_index("s")
    nbase = c * HALF
    iota = lax.iota(jnp.int32, 16)
    zero16 = jnp.zeros((16,), jnp.float32)

    def zrow(i, _):
      zbuf[i, pl.ds(0, 16)] = zero16
      zbuf[i, pl.ds(16, 16)] = zero16
      return 0
    lax.fori_loop(0, ZROWS, zrow, 0)

    nz = (ZSUB - s + NS - 1) // NS
    def zbody(i, _):
      j = s + i * NS
      pltpu.async_copy(zbuf, acc.at[pl.ds(j * ZROWS, ZROWS)], sem_s)
      return 0
    lax.fori_loop(0, nz, zbody, 0)
    def zdrain(i, _):
      pltpu.make_async_copy(zbuf, acc.at[pl.ds(s * ZROWS, ZROWS)], sem_s).wait()
      return 0
    lax.fori_loop(0, nz, zdrain, 0)
    plsc.subcore_barrier()

    nn = (NODE_FULL - s + NS - 1) // NS

    def fire_inputs_n(j, b):
      for f in range(NODE_FIELDS):
        pltpu.async_copy(xT_h.at[pl.ds(f * N + nbase + j * SUB, SUB)],
                         xbuf.at[b, f], sem_in)

    fire_inputs_n(s, 0)

    def nbody(i, _):
      j = s + i * NS
      b = i & 1
      p0, p1 = 2 * b, 2 * b + 1
      for f in range(NODE_FIELDS):
        pltpu.make_async_copy(xT_h.at[pl.ds(f * N + nbase + j * SUB, SUB)],
                              xbuf.at[b, f], sem_in).wait()

      @pl.when(i + 1 < nn)
      def _prefetch():
        fire_inputs_n(j + NS, 1 - b)

      lbase = j * SUB
      for kk in range(SUB // 16):
        lin[b, pl.ds(kk * 16, 16)] = lbase + kk * 16 + iota

      for w in range(NODE_FIELDS // 2):
        add = w > 0
        gcps = [pltpu.async_copy(node_tbl.at[2 * w].at[xbuf.at[b, 2 * w]],
                                 rows.at[p0], sem_g0, add=add),
                pltpu.async_copy(node_tbl.at[2 * w + 1].at[xbuf.at[b, 2 * w + 1]],
                                 rows.at[p1], sem_g0, add=add)]
        for cp in gcps:
          cp.wait()
      @pl.when(i >= 1)
      def _drain_prev():
        for p in (2 * (1 - b), 2 * (1 - b) + 1):
          pltpu.make_async_copy(rows.at[p], acc.at[lin.at[1 - b]], sem_s).wait()
      for p in (p0, p1):
        pltpu.async_copy(rows.at[p], acc.at[lin.at[b]], sem_s, add=True)
      return 0
    lax.fori_loop(0, nn, nbody, 0)

    blast_n = (nn - 1) & 1
    for p in (2 * blast_n, 2 * blast_n + 1):
      pltpu.make_async_copy(rows.at[p], acc.at[lin.at[blast_n]], sem_s).wait()

    @pl.when(s == 0)
    def _node_tail():
      for kk in range(NODE_TAIL // 16):
        lin_tail[pl.ds(kk * 16, 16)] = NODE_TAIL_BASE + kk * 16 + iota
      gbase = nbase + NODE_TAIL_BASE
      cps = [pltpu.async_copy(xT_h.at[pl.ds(f * N + gbase, NODE_TAIL)],
                              xbuf.at[0, f, pl.ds(0, NODE_TAIL)], sem_in)
             for f in range(NODE_FIELDS)]
      for cp in cps:
        cp.wait()
      for w in range(NODE_FIELDS // 2):
        gcps = [pltpu.async_copy(
            node_tbl.at[2 * w + q].at[xbuf.at[0, 2 * w + q, pl.ds(0, NODE_TAIL)]],
            rows.at[q, pl.ds(0, NODE_TAIL)], sem_g0, add=w > 0) for q in (0, 1)]
        for cp in gcps:
          cp.wait()
      for q in (0, 1):
        pltpu.sync_copy(rows.at[q, pl.ds(0, NODE_TAIL)], acc.at[lin_tail],
                        add=True)

    ne = (ESUB - s + NS - 1) // NS

    def fire_inputs_e(j, r):
      pltpu.async_copy(attr_h.at[pl.ds(j * EDGE_FIELDS, EDGE_FIELDS)],
                       abuf.at[r], sem_in)
      pltpu.async_copy(ei_h.at[pl.ds(E + j * SUB, SUB)], dbuf.at[r], sem_in)

    def wait_inputs_e(j, r):
      pltpu.make_async_copy(attr_h.at[pl.ds(j * EDGE_FIELDS, EDGE_FIELDS)],
                            abuf.at[r], sem_in).wait()
      pltpu.make_async_copy(ei_h.at[pl.ds(E + j * SUB, SUB)], dbuf.at[r],
                            sem_in).wait()

    def fire_wave0_e(r):
      pltpu.async_copy(edge_tbl.at[0].at[abuf.at[r, 0]], rows.at[2 * r], sem_g0)
      pltpu.async_copy(edge_tbl.at[1].at[abuf.at[r, 1]], rows.at[2 * r + 1],
                       sem_g0)

    def wait_wave0_e(r):
      for q in (0, 1):
        pltpu.make_async_copy(edge_tbl.at[q].at[abuf.at[r, q]],
                              rows.at[2 * r + q], sem_g0).wait()

    def wait_wave1_e(r):
      for q in (0, 1):
        pltpu.make_async_copy(edge_tbl.at[q].at[abuf.at[r, q]],
                              rows.at[2 * r + q], sem_g1).wait()

    def fire_wave1_e(r):
      pltpu.async_copy(edge_tbl.at[2].at[abuf.at[r, 2]], rows.at[2 * r],
                       sem_g1, add=True)
      pltpu.async_copy(edge_tbl.at[3].at[abuf.at[r, 3]], rows.at[2 * r + 1],
                       sem_g1, add=True)

    def fire_scatter_e(r):
      for q in (0, 1):
        pltpu.async_copy(rows.at[2 * r + q], acc.at[dloc.at[r]], sem_s,
                         add=True)

    def drain_scatter_e(r):
      for q in (0, 1):
        pltpu.make_async_copy(rows.at[2 * r + q], acc.at[dloc.at[r]],
                              sem_s).wait()

    fire_inputs_e(s, 0)
    fire_inputs_e(s + NS, 1)
    wait_inputs_e(s, 0)
    fire_wave0_e(0)

    def ebody(i, _):
      j = s + i * NS
      r = lax.rem(i, 3)
      r_prev = lax.rem(i + 2, 3)
      r_next = lax.rem(i + 1, 3)

      @pl.when(i + 1 < ne)
      def _wait_next_inputs():
        wait_inputs_e(j + NS, r_next)

      for kk in range(SUB // 16):
        v = dbuf[r, pl.ds(kk * 16, 16)]
        loc = v - nbase
        ok = (v >= nbase) & (loc < HALF)
        dloc[r, pl.ds(kk * 16, 16)] = jnp.where(ok, loc, TRASH + kk * 16 + iota)

      @pl.when(i >= 2)
      def _drain():
        drain_scatter_e(r_next)

      @pl.when(i >= 1)
      def _scatter_prev():
        wait_wave1_e(r_prev)
        fire_scatter_e(r_prev)

      @pl.when(i + 2 < ne)
      def _prefetch():
        fire_inputs_e(j + 2 * NS, r_prev)

      wait_wave0_e(r)
      fire_wave1_e(r)

      @pl.when(i + 1 < ne)
      def _next_wave0():
        fire_wave0_e(r_next)
      return 0
    lax.fori_loop(0, ne, ebody, 0)

    blast = ne - 1
    rl = lax.rem(blast, 3)
    drain_scatter_e(lax.rem(blast + 2, 3))
    wait_wave1_e(rl)
    fire_scatter_e(rl)
    drain_scatter_e(rl)

    plsc.subcore_barrier()

    def xbody(i, _):
      j = s + i * NS
      pltpu.async_copy(acc.at[pl.ds(j * SUB, SUB)],
                       out.at[pl.ds(nbase + j * SUB, SUB)], sem_s)
      return 0
    lax.fori_loop(0, nn, xbody, 0)
    def xdrain(i, _):
      pltpu.make_async_copy(acc.at[pl.ds(s * SUB, SUB)],
                            out.at[pl.ds(nbase + s * SUB, SUB)], sem_s).wait()
      return 0
    lax.fori_loop(0, nn, xdrain, 0)

    @pl.when(s == 0)
    def _exp_tail():
      pltpu.sync_copy(acc.at[pl.ds(NODE_TAIL_BASE, NODE_TAIL)],
                      out.at[pl.ds(nbase + NODE_TAIL_BASE, NODE_TAIL)])

  return k(node_tables, edge_tables, xT, attr_i, ei_flat)


MLP_BLK = 2000


def _mlp(agg, W_enc, W_dec):
  def body(a_ref, we_ref, wd_ref, o_ref):
    h = jnp.maximum(
        jnp.dot(a_ref[...], we_ref[...], preferred_element_type=jnp.float32),
        0.0)
    o_ref[...] = jnp.dot(h, wd_ref[...], preferred_element_type=jnp.float32)

  return pl.pallas_call(
      body,
      grid=(N // MLP_BLK,),
      in_specs=[
          pl.BlockSpec((MLP_BLK, D), lambda i: (i, 0)),
          pl.BlockSpec((D, H), lambda i: (0, 0)),
          pl.BlockSpec((H, D), lambda i: (0, 0)),
      ],
      out_specs=pl.BlockSpec((MLP_BLK, D), lambda i: (i, 0)),
      out_shape=jax.ShapeDtypeStruct((N, D), jnp.float32),
  )(agg, W_enc, W_dec)


def kernel(x, edge_attr, edge_index, node_tables, edge_tables, W_enc, W_dec):
  xT = jnp.transpose(x).reshape(-1)
  attr_i = edge_attr.reshape(ESUB, SUB, EDGE_FIELDS).transpose(0, 2, 1)
  attr_i = attr_i.reshape(ESUB * EDGE_FIELDS, SUB)
  ei_flat = edge_index.reshape(-1)
  agg = _sc_embed_aggregate(node_tables, edge_tables, xT, attr_i, ei_flat)
  return _mlp(agg, W_enc, W_dec)


# speedup vs baseline: 2.1896x; 1.0003x over previous
import functools

import jax
import jax.numpy as jnp
from jax import lax
from jax.experimental import pallas as pl
from jax.experimental.pallas import tpu as pltpu
from jax.experimental.pallas import tpu_sc as plsc

N = 100000
E = 1600000
NODE_FIELDS = 8
EDGE_FIELDS = 4
D = 32
H = 64

NC = 2
NS = 16
SUB = 128

HALF = N // NC
TRASH = HALF
ACC_ROWS = 50176
NODE_FULL = HALF // SUB
NODE_TAIL = HALF - NODE_FULL * SUB
NODE_TAIL_BASE = NODE_FULL * SUB
ESUB = E // SUB
ZROWS = 16
ZSUB = ACC_ROWS // ZROWS


def _sc_embed_aggregate(node_tables, edge_tables, xT, attr_i, ei_flat):
  mesh = plsc.VectorSubcoreMesh(
      core_axis_name="c", subcore_axis_name="s", num_cores=NC, num_subcores=NS)

  @functools.partial(
      pl.kernel,
      out_type=jax.ShapeDtypeStruct((N, D), jnp.float32),
      mesh=mesh,
      compiler_params=pltpu.CompilerParams(use_tc_tiling_on_sc=False),
      scratch_types=[
          pltpu.VMEM_SHARED((ACC_ROWS, D), jnp.float32),
          pltpu.VMEM((ZROWS, D), jnp.float32),
          pltpu.VMEM((2, NODE_FIELDS, SUB), jnp.int32),
          pltpu.VMEM((3, EDGE_FIELDS, SUB), jnp.int32),
          pltpu.VMEM((3, SUB), jnp.int32),
          pltpu.VMEM((3, SUB), jnp.int32),
          pltpu.VMEM((2, SUB), jnp.int32),
          pltpu.VMEM((NODE_TAIL,), jnp.int32),
          pltpu.VMEM((6, SUB, D), jnp.float32),
          pltpu.SemaphoreType.DMA,
          pltpu.SemaphoreType.DMA,
          pltpu.SemaphoreType.DMA,
          pltpu.SemaphoreType.DMA,
      ],
  )
  def k(node_tbl, edge_tbl, xT_h, attr_h, ei_h, out, acc, zbuf, xbuf, abuf,
        dbuf, dloc, lin, lin_tail, rows, sem_in, sem_g0, sem_g1, sem_s):
    c = lax.axis_index("c")
    s = lax.axis_index("s")
    nbase = c * HALF
    iota = lax.iota(jnp.int32, 16)
    zero16 = jnp.zeros((16,), jnp.float32)

    def zrow(i, _):
      zbuf[i, pl.ds(0, 16)] = zero16
      zbuf[i, pl.ds(16, 16)] = zero16
      return 0
    lax.fori_loop(0, ZROWS, zrow, 0)

    nz = (ZSUB - s + NS - 1) // NS
    def zbody(i, _):
      j = s + i * NS
      pltpu.async_copy(zbuf, acc.at[pl.ds(j * ZROWS, ZROWS)], sem_s)
      return 0
    lax.fori_loop(0, nz, zbody, 0)
    def zdrain(i, _):
      pltpu.make_async_copy(zbuf, acc.at[pl.ds(s * ZROWS, ZROWS)], sem_s).wait()
      return 0
    lax.fori_loop(0, nz, zdrain, 0)
    plsc.subcore_barrier()

    nn = (NODE_FULL - s + NS - 1) // NS

    def fire_inputs_n(j, b):
      for f in range(NODE_FIELDS):
        pltpu.async_copy(xT_h.at[pl.ds(f * N + nbase + j * SUB, SUB)],
                         xbuf.at[b, f], sem_in)

    fire_inputs_n(s, 0)

    def nbody(i, _):
      j = s + i * NS
      b = i & 1
      p0, p1 = 2 * b, 2 * b + 1
      for f in range(NODE_FIELDS):
        pltpu.make_async_copy(xT_h.at[pl.ds(f * N + nbase + j * SUB, SUB)],
                              xbuf.at[b, f], sem_in).wait()

      @pl.when(i + 1 < nn)
      def _prefetch():
        fire_inputs_n(j + NS, 1 - b)

      lbase = j * SUB
      for kk in range(SUB // 16):
        lin[b, pl.ds(kk * 16, 16)] = lbase + kk * 16 + iota

      for w in range(NODE_FIELDS // 2):
        add = w > 0
        gcps = [pltpu.async_copy(node_tbl.at[2 * w].at[xbuf.at[b, 2 * w]],
                                 rows.at[p0], sem_g0, add=add),
                pltpu.async_copy(node_tbl.at[2 * w + 1].at[xbuf.at[b, 2 * w + 1]],
                                 rows.at[p1], sem_g0, add=add)]
        for cp in gcps:
          cp.wait()
      @pl.when(i >= 1)
      def _drain_prev():
        for p in (2 * (1 - b), 2 * (1 - b) + 1):
          pltpu.make_async_copy(rows.at[p], acc.at[lin.at[1 - b]], sem_s).wait()
      for p in (p0, p1):
        pltpu.async_copy(rows.at[p], acc.at[lin.at[b]], sem_s, add=True)
      return 0
    lax.fori_loop(0, nn, nbody, 0)

    blast_n = (nn - 1) & 1
    for p in (2 * blast_n, 2 * blast_n + 1):
      pltpu.make_async_copy(rows.at[p], acc.at[lin.at[blast_n]], sem_s).wait()

    @pl.when(s == 0)
    def _node_tail():
      for kk in range(NODE_TAIL // 16):
        lin_tail[pl.ds(kk * 16, 16)] = NODE_TAIL_BASE + kk * 16 + iota
      gbase = nbase + NODE_TAIL_BASE
      cps = [pltpu.async_copy(xT_h.at[pl.ds(f * N + gbase, NODE_TAIL)],
                              xbuf.at[0, f, pl.ds(0, NODE_TAIL)], sem_in)
             for f in range(NODE_FIELDS)]
      for cp in cps:
        cp.wait()
      for w in range(NODE_FIELDS // 2):
        gcps = [pltpu.async_copy(
            node_tbl.at[2 * w + q].at[xbuf.at[0, 2 * w + q, pl.ds(0, NODE_TAIL)]],
            rows.at[q, pl.ds(0, NODE_TAIL)], sem_g0, add=w > 0) for q in (0, 1)]
        for cp in gcps:
          cp.wait()
      for q in (0, 1):
        pltpu.sync_copy(rows.at[q, pl.ds(0, NODE_TAIL)], acc.at[lin_tail],
                        add=True)

    ne = (ESUB - s + NS - 1) // NS

    def fire_inputs_e(j, r):
      pltpu.async_copy(attr_h.at[pl.ds(j * EDGE_FIELDS, EDGE_FIELDS)],
                       abuf.at[r], sem_in)
      pltpu.async_copy(ei_h.at[pl.ds(E + j * SUB, SUB)], dbuf.at[r], sem_in)

    def wait_inputs_e(j, r):
      pltpu.make_async_copy(attr_h.at[pl.ds(j * EDGE_FIELDS, EDGE_FIELDS)],
                            abuf.at[r], sem_in).wait()
      pltpu.make_async_copy(ei_h.at[pl.ds(E + j * SUB, SUB)], dbuf.at[r],
                            sem_in).wait()

    def fire_wave0_e(r):
      pltpu.async_copy(edge_tbl.at[0].at[abuf.at[r, 0]], rows.at[2 * r], sem_g0)
      pltpu.async_copy(edge_tbl.at[1].at[abuf.at[r, 1]], rows.at[2 * r + 1],
                       sem_g0)

    def wait_wave0_e(r):
      for q in (0, 1):
        pltpu.make_async_copy(edge_tbl.at[q].at[abuf.at[r, q]],
                              rows.at[2 * r + q], sem_g0).wait()

    def wait_wave1_e(r):
      for q in (0, 1):
        pltpu.make_async_copy(edge_tbl.at[q].at[abuf.at[r, q]],
                              rows.at[2 * r + q], sem_g1).wait()

    def fire_wave1_e(r):
      pltpu.async_copy(edge_tbl.at[2].at[abuf.at[r, 2]], rows.at[2 * r],
                       sem_g1, add=True)
      pltpu.async_copy(edge_tbl.at[3].at[abuf.at[r, 3]], rows.at[2 * r + 1],
                       sem_g1, add=True)

    def fire_scatter_e(r):
      for q in (0, 1):
        pltpu.async_copy(rows.at[2 * r + q], acc.at[dloc.at[r]], sem_s,
                         add=True)

    def drain_scatter_e(r):
      for q in (0, 1):
        pltpu.make_async_copy(rows.at[2 * r + q], acc.at[dloc.at[r]],
                              sem_s).wait()

    fire_inputs_e(s, 0)
    wait_inputs_e(s, 0)
    fire_wave0_e(0)
    fire_inputs_e(s + NS, 1)

    def ebody(i, _):
      j = s + i * NS
      r = lax.rem(i, 3)
      r_prev = lax.rem(i + 2, 3)
      r_next = lax.rem(i + 1, 3)

      @pl.when(i + 1 < ne)
      def _wait_next_inputs():
        wait_inputs_e(j + NS, r_next)

      for kk in range(SUB // 16):
        v = dbuf[r, pl.ds(kk * 16, 16)]
        loc = v - nbase
        ok = (v >= nbase) & (loc < HALF)
        dloc[r, pl.ds(kk * 16, 16)] = jnp.where(ok, loc, TRASH + kk * 16 + iota)

      @pl.when(i >= 2)
      def _drain():
        drain_scatter_e(r_next)

      @pl.when(i >= 1)
      def _scatter_prev():
        wait_wave1_e(r_prev)
        fire_scatter_e(r_prev)

      @pl.when(i + 2 < ne)
      def _prefetch():
        fire_inputs_e(j + 2 * NS, r_prev)

      wait_wave0_e(r)
      fire_wave1_e(r)

      @pl.when(i + 1 < ne)
      def _next_wave0():
        fire_wave0_e(r_next)
      return 0
    lax.fori_loop(0, ne, ebody, 0)

    blast = ne - 1
    rl = lax.rem(blast, 3)
    drain_scatter_e(lax.rem(blast + 2, 3))
    wait_wave1_e(rl)
    fire_scatter_e(rl)
    drain_scatter_e(rl)

    plsc.subcore_barrier()

    def xbody(i, _):
      j = s + i * NS
      pltpu.async_copy(acc.at[pl.ds(j * SUB, SUB)],
                       out.at[pl.ds(nbase + j * SUB, SUB)], sem_s)
      return 0
    lax.fori_loop(0, nn, xbody, 0)
    def xdrain(i, _):
      pltpu.make_async_copy(acc.at[pl.ds(s * SUB, SUB)],
                            out.at[pl.ds(nbase + s * SUB, SUB)], sem_s).wait()
      return 0
    lax.fori_loop(0, nn, xdrain, 0)

    @pl.when(s == 0)
    def _exp_tail():
      pltpu.sync_copy(acc.at[pl.ds(NODE_TAIL_BASE, NODE_TAIL)],
                      out.at[pl.ds(nbase + NODE_TAIL_BASE, NODE_TAIL)])

  return k(node_tables, edge_tables, xT, attr_i, ei_flat)


MLP_BLK = 2000


def _mlp(agg, W_enc, W_dec):
  def body(a_ref, we_ref, wd_ref, o_ref):
    h = jnp.maximum(
        jnp.dot(a_ref[...], we_ref[...], preferred_element_type=jnp.float32),
        0.0)
    o_ref[...] = jnp.dot(h, wd_ref[...], preferred_element_type=jnp.float32)

  return pl.pallas_call(
      body,
      grid=(N // MLP_BLK,),
      in_specs=[
          pl.BlockSpec((MLP_BLK, D), lambda i: (i, 0)),
          pl.BlockSpec((D, H), lambda i: (0, 0)),
          pl.BlockSpec((H, D), lambda i: (0, 0)),
      ],
      out_specs=pl.BlockSpec((MLP_BLK, D), lambda i: (i, 0)),
      out_shape=jax.ShapeDtypeStruct((N, D), jnp.float32),
  )(agg, W_enc, W_dec)


def kernel(x, edge_attr, edge_index, node_tables, edge_tables, W_enc, W_dec):
  xT = jnp.transpose(x).reshape(-1)
  attr_i = edge_attr.reshape(ESUB, SUB, EDGE_FIELDS).transpose(0, 2, 1)
  attr_i = attr_i.reshape(ESUB * EDGE_FIELDS, SUB)
  ei_flat = edge_index.reshape(-1)
  agg = _sc_embed_aggregate(node_tables, edge_tables, xT, attr_i, ei_flat)
  return _mlp(agg, W_enc, W_dec)
